# Initial kernel scaffold; baseline (speedup 1.0000x reference)
import jax
import jax.numpy as jnp
from jax.experimental import pallas as pl


def kernel(traj, mask, w_ih_f, w_hh_f, b_ih_f, b_hh_f, w_ih_b, w_hh_b, b_ih_b, b_hh_b, codebook):
    raise NotImplementedError("write your pallas kernel here")


# speedup vs baseline: 14.9460x; 14.9460x over previous
import jax
import jax.numpy as jnp
from jax.experimental import pallas as pl
from jax.experimental.pallas import tpu as pltpu


def _gru_cell(x, h, gi, whh_ref, bhh_ref, H):
    gh = jnp.dot(h, whh_ref[:], preferred_element_type=jnp.float32) + bhh_ref[:]
    r = jax.nn.sigmoid(gi[:, 0:H] + gh[:, 0:H])
    z = jax.nn.sigmoid(gi[:, H:2 * H] + gh[:, H:2 * H])
    n = jnp.tanh(gi[:, 2 * H:3 * H] + r * gh[:, 2 * H:3 * H])
    return (1.0 - z) * n + z * h


def _bigru_body(x_ref, mask_ref,
                wihf_ref, whhf_ref, bihf_ref, bhhf_ref,
                wihb_ref, whhb_ref, bihb_ref, bhhb_ref,
                z_e_ref, out_f_ref, out_b_ref):
    B, T = mask_ref.shape
    H = whhf_ref.shape[0]

    def step(i, carry):
        hf, hb = carry
        xf = x_ref[pl.ds(i * B, B), :]
        xb = x_ref[pl.ds((T - 1 - i) * B, B), :]
        gif = jnp.dot(xf, wihf_ref[:], preferred_element_type=jnp.float32) + bihf_ref[:]
        gib = jnp.dot(xb, wihb_ref[:], preferred_element_type=jnp.float32) + bihb_ref[:]
        hf = _gru_cell(xf, hf, gif, whhf_ref, bhhf_ref, H)
        hb = _gru_cell(xb, hb, gib, whhb_ref, bhhb_ref, H)
        out_f_ref[pl.ds(i * B, B), :] = hf
        out_b_ref[pl.ds((T - 1 - i) * B, B), :] = hb
        return hf, hb

    h0 = jnp.zeros((B, H), jnp.float32)
    jax.lax.fori_loop(0, T, step, (h0, h0))

    m = mask_ref[:]
    rowmax = jnp.max(m, axis=1, keepdims=True)
    tio = jax.lax.broadcasted_iota(jnp.int32, (B, T), 1)
    tb = jnp.max(jnp.where(m == rowmax, tio, -1), axis=1, keepdims=True)

    cio = jax.lax.broadcasted_iota(jnp.int32, (B, T * B), 1)
    bio = jax.lax.broadcasted_iota(jnp.int32, (B, T * B), 0)
    sel = (cio == tb * B + bio).astype(jnp.float32)
    z_e_ref[:, 0:H] = jnp.dot(sel, out_f_ref[:], preferred_element_type=jnp.float32)
    z_e_ref[:, H:2 * H] = jnp.dot(sel, out_b_ref[:], preferred_element_type=jnp.float32)


def kernel(traj, mask, w_ih_f, w_hh_f, b_ih_f, b_hh_f,
           w_ih_b, w_hh_b, b_ih_b, b_hh_b, codebook):
    del codebook
    B, T, IN = traj.shape
    H = w_hh_f.shape[1]
    x2d = jnp.transpose(traj, (1, 0, 2)).reshape(T * B, IN)
    args = (
        x2d, mask,
        w_ih_f.T, w_hh_f.T, b_ih_f.reshape(1, -1), b_hh_f.reshape(1, -1),
        w_ih_b.T, w_hh_b.T, b_ih_b.reshape(1, -1), b_hh_b.reshape(1, -1),
    )
    return pl.pallas_call(
        _bigru_body,
        out_shape=jax.ShapeDtypeStruct((B, 2 * H), jnp.float32),
        scratch_shapes=[
            pltpu.VMEM((T * B, H), jnp.float32),
            pltpu.VMEM((T * B, H), jnp.float32),
        ],
    )(*args)


# speedup vs baseline: 18.1760x; 1.2161x over previous
import jax
import jax.numpy as jnp
from jax.experimental import pallas as pl
from jax.experimental.pallas import tpu as pltpu


def _bigru_body(x_ref, mask_ref,
                wihf_ref, whhf_ref, bgf_ref, bnf_ref,
                wihb_ref, whhb_ref, bgb_ref, bnb_ref,
                z_e_ref, gif_ref, gib_ref, out_f_ref, out_b_ref):
    B, T = mask_ref.shape
    H = whhf_ref.shape[0]

    gif_ref[:] = jnp.dot(x_ref[:], wihf_ref[:],
                         preferred_element_type=jnp.float32) + bgf_ref[:]
    gib_ref[:] = jnp.dot(x_ref[:], wihb_ref[:],
                         preferred_element_type=jnp.float32) + bgb_ref[:]

    def cell(gi, h, whh_ref, bn_ref):
        gh = jnp.dot(h.astype(jnp.bfloat16), whh_ref[:],
                     preferred_element_type=jnp.float32)
        r = jax.nn.sigmoid(gi[:, 0:H] + gh[:, 0:H])
        z = jax.nn.sigmoid(gi[:, H:2 * H] + gh[:, H:2 * H])
        n = jnp.tanh(gi[:, 2 * H:3 * H] + r * (gh[:, 2 * H:3 * H] + bn_ref[:]))
        return n + z * (h - n)

    def step(i, carry):
        hf, hb = carry
        hf = cell(gif_ref[pl.ds(i * B, B), :], hf, whhf_ref, bnf_ref)
        hb = cell(gib_ref[pl.ds((T - 1 - i) * B, B), :], hb, whhb_ref, bnb_ref)
        out_f_ref[pl.ds(i * B, B), :] = hf
        out_b_ref[pl.ds((T - 1 - i) * B, B), :] = hb
        return hf, hb

    h0 = jnp.zeros((B, H), jnp.float32)
    jax.lax.fori_loop(0, T, step, (h0, h0))

    m = mask_ref[:]
    rowmax = jnp.max(m, axis=1, keepdims=True)
    tio = jax.lax.broadcasted_iota(jnp.int32, (B, T), 1)
    tb = jnp.max(jnp.where(m == rowmax, tio, -1), axis=1, keepdims=True)

    cio = jax.lax.broadcasted_iota(jnp.int32, (B, T * B), 1)
    bio = jax.lax.broadcasted_iota(jnp.int32, (B, T * B), 0)
    sel = (cio == tb * B + bio).astype(jnp.float32)
    z_e_ref[:, 0:H] = jnp.dot(sel, out_f_ref[:], preferred_element_type=jnp.float32)
    z_e_ref[:, H:2 * H] = jnp.dot(sel, out_b_ref[:], preferred_element_type=jnp.float32)


def kernel(traj, mask, w_ih_f, w_hh_f, b_ih_f, b_hh_f,
           w_ih_b, w_hh_b, b_ih_b, b_hh_b, codebook):
    del codebook
    B, T, IN = traj.shape
    H = w_hh_f.shape[1]
    x2d = jnp.transpose(traj, (1, 0, 2)).reshape(T * B, IN).astype(jnp.bfloat16)
    zeros_h = jnp.zeros((H,), jnp.float32)
    bg_f = (b_ih_f + jnp.concatenate([b_hh_f[:2 * H], zeros_h])).reshape(1, -1)
    bg_b = (b_ih_b + jnp.concatenate([b_hh_b[:2 * H], zeros_h])).reshape(1, -1)
    args = (
        x2d, mask,
        w_ih_f.T.astype(jnp.bfloat16), w_hh_f.T.astype(jnp.bfloat16),
        bg_f, b_hh_f[2 * H:].reshape(1, -1),
        w_ih_b.T.astype(jnp.bfloat16), w_hh_b.T.astype(jnp.bfloat16),
        bg_b, b_hh_b[2 * H:].reshape(1, -1),
    )
    return pl.pallas_call(
        _bigru_body,
        out_shape=jax.ShapeDtypeStruct((B, 2 * H), jnp.float32),
        scratch_shapes=[
            pltpu.VMEM((T * B, 3 * H), jnp.float32),
            pltpu.VMEM((T * B, 3 * H), jnp.float32),
            pltpu.VMEM((T * B, H), jnp.float32),
            pltpu.VMEM((T * B, H), jnp.float32),
        ],
    )(*args)


# speedup vs baseline: 20.2867x; 1.1161x over previous
import jax
import jax.numpy as jnp
from jax.experimental import pallas as pl
from jax.experimental.pallas import tpu as pltpu


def _bigru_body(x_ref, mask_ref,
                wihf_ref, whhf_ref, bgf_ref, bnf_ref,
                wihb_ref, whhb_ref, bgb_ref, bnb_ref,
                z_e_ref, gif_ref, gib_ref, out_f_ref, out_b_ref):
    B, T = mask_ref.shape
    H = whhf_ref.shape[0]

    gif_ref[:] = jnp.dot(x_ref[:], wihf_ref[:],
                         preferred_element_type=jnp.float32) + bgf_ref[:]
    gib_ref[:] = jnp.dot(x_ref[:], wihb_ref[:],
                         preferred_element_type=jnp.float32) + bgb_ref[:]

    def sig(a):
        return 0.5 * jnp.tanh(0.5 * a) + 0.5

    def cell(gi, h, whh_ref, bn_ref):
        gh = jnp.dot(h.astype(jnp.bfloat16), whh_ref[:],
                     preferred_element_type=jnp.float32)
        r = sig(gi[:, 0:H] + gh[:, 0:H])
        z = sig(gi[:, H:2 * H] + gh[:, H:2 * H])
        n = jnp.tanh(gi[:, 2 * H:3 * H] + r * (gh[:, 2 * H:3 * H] + bn_ref[:]))
        return n + z * (h - n)

    def step(i, carry):
        hf, hb = carry
        hf = cell(gif_ref[pl.ds(i * B, B), :], hf, whhf_ref, bnf_ref)
        hb = cell(gib_ref[pl.ds((T - 1 - i) * B, B), :], hb, whhb_ref, bnb_ref)
        out_f_ref[pl.ds(i * B, B), :] = hf
        out_b_ref[pl.ds((T - 1 - i) * B, B), :] = hb
        return hf, hb

    h0 = jnp.zeros((B, H), jnp.float32)
    jax.lax.fori_loop(0, T, step, (h0, h0), unroll=2)

    m = mask_ref[:]
    rowmax = jnp.max(m, axis=1, keepdims=True)
    tio = jax.lax.broadcasted_iota(jnp.int32, (B, T), 1)
    tb = jnp.max(jnp.where(m == rowmax, tio, -1), axis=1, keepdims=True)

    cio = jax.lax.broadcasted_iota(jnp.int32, (B, T * B), 1)
    bio = jax.lax.broadcasted_iota(jnp.int32, (B, T * B), 0)
    sel = (cio == tb * B + bio).astype(jnp.float32)
    z_e_ref[:, 0:H] = jnp.dot(sel, out_f_ref[:], preferred_element_type=jnp.float32)
    z_e_ref[:, H:2 * H] = jnp.dot(sel, out_b_ref[:], preferred_element_type=jnp.float32)


def kernel(traj, mask, w_ih_f, w_hh_f, b_ih_f, b_hh_f,
           w_ih_b, w_hh_b, b_ih_b, b_hh_b, codebook):
    del codebook
    B, T, IN = traj.shape
    H = w_hh_f.shape[1]
    x2d = jnp.transpose(traj, (1, 0, 2)).reshape(T * B, IN).astype(jnp.bfloat16)
    zeros_h = jnp.zeros((H,), jnp.float32)
    bg_f = (b_ih_f + jnp.concatenate([b_hh_f[:2 * H], zeros_h])).reshape(1, -1)
    bg_b = (b_ih_b + jnp.concatenate([b_hh_b[:2 * H], zeros_h])).reshape(1, -1)
    args = (
        x2d, mask,
        w_ih_f.T.astype(jnp.bfloat16), w_hh_f.T.astype(jnp.bfloat16),
        bg_f, b_hh_f[2 * H:].reshape(1, -1),
        w_ih_b.T.astype(jnp.bfloat16), w_hh_b.T.astype(jnp.bfloat16),
        bg_b, b_hh_b[2 * H:].reshape(1, -1),
    )
    return pl.pallas_call(
        _bigru_body,
        out_shape=jax.ShapeDtypeStruct((B, 2 * H), jnp.float32),
        scratch_shapes=[
            pltpu.VMEM((T * B, 3 * H), jnp.float32),
            pltpu.VMEM((T * B, 3 * H), jnp.float32),
            pltpu.VMEM((T * B, H), jnp.float32),
            pltpu.VMEM((T * B, H), jnp.float32),
        ],
    )(*args)


# speedup vs baseline: 21.5525x; 1.0624x over previous
import jax
import jax.numpy as jnp
from jax.experimental import pallas as pl
from jax.experimental.pallas import tpu as pltpu


def _bigru_body(x_ref, mask_ref, wih_ref, bg_ref,
                whhf_ref, bnf_ref, whhb_ref, bnb_ref,
                z_e_ref, gi_ref, out_ref):
    B, T = mask_ref.shape
    H = whhf_ref.shape[0]

    gi_ref[:] = (jnp.dot(x_ref[:], wih_ref[:],
                         preferred_element_type=jnp.float32)
                 + bg_ref[:]).astype(jnp.bfloat16)

    def sig(a):
        return 0.5 * jnp.tanh(0.5 * a) + 0.5

    def cell(gi, h, whh_ref, bn_ref):
        gi = gi.astype(jnp.float32)
        gh = jnp.dot(h.astype(jnp.bfloat16), whh_ref[:],
                     preferred_element_type=jnp.float32)
        r = sig(gi[:, 0:H] + gh[:, 0:H])
        z = sig(gi[:, H:2 * H] + gh[:, H:2 * H])
        n = jnp.tanh(gi[:, 2 * H:3 * H] + r * (gh[:, 2 * H:3 * H] + bn_ref[:]))
        return n + z * (h - n)

    def step(i, carry):
        hf, hb = carry
        hf = cell(gi_ref[pl.ds(i * B, B), 0:3 * H], hf, whhf_ref, bnf_ref)
        hb = cell(gi_ref[pl.ds((T - 1 - i) * B, B), 3 * H:6 * H], hb,
                  whhb_ref, bnb_ref)
        out_ref[pl.ds(i * B, B), 0:H] = hf
        out_ref[pl.ds((T - 1 - i) * B, B), H:2 * H] = hb
        return hf, hb

    h0 = jnp.zeros((B, H), jnp.float32)
    jax.lax.fori_loop(0, T, step, (h0, h0), unroll=4)

    m = mask_ref[:]
    rowmax = jnp.max(m, axis=1, keepdims=True)
    tio = jax.lax.broadcasted_iota(jnp.int32, (B, T), 1)
    tb = jnp.max(jnp.where(m == rowmax, tio, -1), axis=1, keepdims=True)

    cio = jax.lax.broadcasted_iota(jnp.int32, (B, T * B), 1)
    bio = jax.lax.broadcasted_iota(jnp.int32, (B, T * B), 0)
    sel = (cio == tb * B + bio).astype(jnp.float32)
    z_e_ref[:] = jnp.dot(sel, out_ref[:], preferred_element_type=jnp.float32)


def kernel(traj, mask, w_ih_f, w_hh_f, b_ih_f, b_hh_f,
           w_ih_b, w_hh_b, b_ih_b, b_hh_b, codebook):
    del codebook
    B, T, IN = traj.shape
    H = w_hh_f.shape[1]
    x2d = jnp.transpose(traj, (1, 0, 2)).reshape(T * B, IN).astype(jnp.bfloat16)
    wih = jnp.concatenate([w_ih_f.T, w_ih_b.T], axis=1).astype(jnp.bfloat16)
    zeros_h = jnp.zeros((H,), jnp.float32)
    bg = jnp.concatenate([
        b_ih_f + jnp.concatenate([b_hh_f[:2 * H], zeros_h]),
        b_ih_b + jnp.concatenate([b_hh_b[:2 * H], zeros_h]),
    ]).reshape(1, -1)
    args = (
        x2d, mask, wih, bg,
        w_hh_f.T.astype(jnp.bfloat16), b_hh_f[2 * H:].reshape(1, -1),
        w_hh_b.T.astype(jnp.bfloat16), b_hh_b[2 * H:].reshape(1, -1),
    )
    return pl.pallas_call(
        _bigru_body,
        out_shape=jax.ShapeDtypeStruct((B, 2 * H), jnp.float32),
        scratch_shapes=[
            pltpu.VMEM((T * B, 6 * H), jnp.bfloat16),
            pltpu.VMEM((T * B, 2 * H), jnp.float32),
        ],
    )(*args)


# speedup vs baseline: 22.0680x; 1.0239x over previous
import jax
import jax.numpy as jnp
from jax.experimental import pallas as pl
from jax.experimental.pallas import tpu as pltpu


def _bigru_body(x_ref, mask_ref, wih_ref, bg_ref,
                whhf_ref, bnf_ref, whhb_ref, bnb_ref,
                z_e_ref, gi_ref, out_ref):
    B, T = mask_ref.shape
    H = whhf_ref.shape[0]

    gi_ref[:] = (jnp.dot(x_ref[:], wih_ref[:],
                         preferred_element_type=jnp.float32)
                 + bg_ref[:]).astype(jnp.bfloat16)

    def sig(a):
        return 0.5 * jnp.tanh(0.5 * a) + 0.5

    def cell(gi, h, whh_ref, bn_ref):
        gi = gi.astype(jnp.float32)
        gh = jnp.dot(h.astype(jnp.bfloat16), whh_ref[:],
                     preferred_element_type=jnp.float32)
        r = sig(gi[:, 0:H] + gh[:, 0:H])
        z = sig(gi[:, H:2 * H] + gh[:, H:2 * H])
        n = jnp.tanh(gi[:, 2 * H:3 * H] + r * (gh[:, 2 * H:3 * H] + bn_ref[:]))
        return n + z * (h - n)

    def step(i, carry):
        hf, hb = carry
        hf = cell(gi_ref[pl.ds(i * B, B), 0:3 * H], hf, whhf_ref, bnf_ref)
        hb = cell(gi_ref[pl.ds((T - 1 - i) * B, B), 3 * H:6 * H], hb,
                  whhb_ref, bnb_ref)
        out_ref[pl.ds(i * B, B), 0:H] = hf
        out_ref[pl.ds((T - 1 - i) * B, B), H:2 * H] = hb
        return hf, hb

    h0 = jnp.zeros((B, H), jnp.float32)
    jax.lax.fori_loop(0, T, step, (h0, h0), unroll=8)

    m = mask_ref[:]
    rowmax = jnp.max(m, axis=1, keepdims=True)
    tio = jax.lax.broadcasted_iota(jnp.int32, (B, T), 1)
    tb = jnp.max(jnp.where(m == rowmax, tio, -1), axis=1, keepdims=True)

    cio = jax.lax.broadcasted_iota(jnp.int32, (B, T * B), 1)
    bio = jax.lax.broadcasted_iota(jnp.int32, (B, T * B), 0)
    sel = (cio == tb * B + bio).astype(jnp.float32)
    z_e_ref[:] = jnp.dot(sel, out_ref[:], preferred_element_type=jnp.float32)


def kernel(traj, mask, w_ih_f, w_hh_f, b_ih_f, b_hh_f,
           w_ih_b, w_hh_b, b_ih_b, b_hh_b, codebook):
    del codebook
    B, T, IN = traj.shape
    H = w_hh_f.shape[1]
    x2d = jnp.transpose(traj, (1, 0, 2)).reshape(T * B, IN).astype(jnp.bfloat16)
    wih = jnp.concatenate([w_ih_f.T, w_ih_b.T], axis=1).astype(jnp.bfloat16)
    zeros_h = jnp.zeros((H,), jnp.float32)
    bg = jnp.concatenate([
        b_ih_f + jnp.concatenate([b_hh_f[:2 * H], zeros_h]),
        b_ih_b + jnp.concatenate([b_hh_b[:2 * H], zeros_h]),
    ]).reshape(1, -1)
    args = (
        x2d, mask, wih, bg,
        w_hh_f.T.astype(jnp.bfloat16), b_hh_f[2 * H:].reshape(1, -1),
        w_hh_b.T.astype(jnp.bfloat16), b_hh_b[2 * H:].reshape(1, -1),
    )
    return pl.pallas_call(
        _bigru_body,
        out_shape=jax.ShapeDtypeStruct((B, 2 * H), jnp.float32),
        scratch_shapes=[
            pltpu.VMEM((T * B, 6 * H), jnp.bfloat16),
            pltpu.VMEM((T * B, 2 * H), jnp.float32),
        ],
    )(*args)


# speedup vs baseline: 22.3867x; 1.0144x over previous
import jax
import jax.numpy as jnp
from jax.experimental import pallas as pl
from jax.experimental.pallas import tpu as pltpu


def _bigru_body(x_ref, mask_ref, wih_ref, bg_ref,
                whhf_ref, bnf_ref, whhb_ref, bnb_ref,
                z_e_ref, gi_ref, out_ref):
    B, T = mask_ref.shape
    H = whhf_ref.shape[0]

    gi_ref[:] = (jnp.dot(x_ref[:], wih_ref[:],
                         preferred_element_type=jnp.float32)
                 + bg_ref[:]).astype(jnp.bfloat16)

    def sig(a):
        return 0.5 * jnp.tanh(0.5 * a) + 0.5

    def cell(gi, h, whh_ref, bn_ref):
        gi = gi.astype(jnp.float32)
        gh = jnp.dot(h.astype(jnp.bfloat16), whh_ref[:],
                     preferred_element_type=jnp.float32)
        r = sig(gi[:, 0:H] + gh[:, 0:H])
        z = sig(gi[:, H:2 * H] + gh[:, H:2 * H])
        n = jnp.tanh(gi[:, 2 * H:3 * H] + r * (gh[:, 2 * H:3 * H] + bn_ref[:]))
        return n + z * (h - n)

    def step(i, carry):
        hf, hb = carry
        hf = cell(gi_ref[pl.ds(i * B, B), 0:3 * H], hf, whhf_ref, bnf_ref)
        hb = cell(gi_ref[pl.ds((T - 1 - i) * B, B), 3 * H:6 * H], hb,
                  whhb_ref, bnb_ref)
        out_ref[pl.ds(i * B, B), 0:H] = hf
        out_ref[pl.ds((T - 1 - i) * B, B), H:2 * H] = hb
        return hf, hb

    h0 = jnp.zeros((B, H), jnp.float32)
    jax.lax.fori_loop(0, T, step, (h0, h0), unroll=16)

    m = mask_ref[:]
    rowmax = jnp.max(m, axis=1, keepdims=True)
    tio = jax.lax.broadcasted_iota(jnp.int32, (B, T), 1)
    tb = jnp.max(jnp.where(m == rowmax, tio, -1), axis=1, keepdims=True)

    cio = jax.lax.broadcasted_iota(jnp.int32, (B, T * B), 1)
    bio = jax.lax.broadcasted_iota(jnp.int32, (B, T * B), 0)
    sel = (cio == tb * B + bio).astype(jnp.float32)
    z_e_ref[:] = jnp.dot(sel, out_ref[:], preferred_element_type=jnp.float32)


def kernel(traj, mask, w_ih_f, w_hh_f, b_ih_f, b_hh_f,
           w_ih_b, w_hh_b, b_ih_b, b_hh_b, codebook):
    del codebook
    B, T, IN = traj.shape
    H = w_hh_f.shape[1]
    x2d = jnp.transpose(traj, (1, 0, 2)).reshape(T * B, IN).astype(jnp.bfloat16)
    wih = jnp.concatenate([w_ih_f.T, w_ih_b.T], axis=1).astype(jnp.bfloat16)
    zeros_h = jnp.zeros((H,), jnp.float32)
    bg = jnp.concatenate([
        b_ih_f + jnp.concatenate([b_hh_f[:2 * H], zeros_h]),
        b_ih_b + jnp.concatenate([b_hh_b[:2 * H], zeros_h]),
    ]).reshape(1, -1)
    args = (
        x2d, mask, wih, bg,
        w_hh_f.T.astype(jnp.bfloat16), b_hh_f[2 * H:].reshape(1, -1),
        w_hh_b.T.astype(jnp.bfloat16), b_hh_b[2 * H:].reshape(1, -1),
    )
    return pl.pallas_call(
        _bigru_body,
        out_shape=jax.ShapeDtypeStruct((B, 2 * H), jnp.float32),
        scratch_shapes=[
            pltpu.VMEM((T * B, 6 * H), jnp.bfloat16),
            pltpu.VMEM((T * B, 2 * H), jnp.float32),
        ],
    )(*args)


# speedup vs baseline: 22.6123x; 1.0101x over previous
import jax
import jax.numpy as jnp
from jax.experimental import pallas as pl
from jax.experimental.pallas import tpu as pltpu


def _bigru_body(x_ref, mask_ref, wih_ref, bg_ref,
                whhf_ref, bnf_ref, whhb_ref, bnb_ref,
                z_e_ref, gi_ref, out_ref):
    B, T = mask_ref.shape
    H = whhf_ref.shape[0]

    gi_ref[:] = (jnp.dot(x_ref[:], wih_ref[:],
                         preferred_element_type=jnp.float32)
                 + bg_ref[:]).astype(jnp.bfloat16)

    def cell(gi, h, whh_ref, bn_ref):
        gi = gi.astype(jnp.float32)
        gh = jnp.dot(h.astype(jnp.bfloat16), whh_ref[:],
                     preferred_element_type=jnp.float32)
        tr = jnp.tanh(gi[:, 0:H] + 0.5 * gh[:, 0:H])
        tz = jnp.tanh(gi[:, H:2 * H] + 0.5 * gh[:, H:2 * H])
        hx = 0.5 * gh[:, 2 * H:3 * H] + bn_ref[:]
        n = jnp.tanh((gi[:, 2 * H:3 * H] + hx) + hx * tr)
        z = 0.5 * tz + 0.5
        return n + z * (h - n)

    def step(i, carry):
        hf, hb = carry
        hf = cell(gi_ref[pl.ds(i * B, B), 0:3 * H], hf, whhf_ref, bnf_ref)
        hb = cell(gi_ref[pl.ds((T - 1 - i) * B, B), 3 * H:6 * H], hb,
                  whhb_ref, bnb_ref)
        out_ref[pl.ds(i * B, B), 0:H] = hf
        out_ref[pl.ds((T - 1 - i) * B, B), H:2 * H] = hb
        return hf, hb

    h0 = jnp.zeros((B, H), jnp.float32)
    jax.lax.fori_loop(0, T, step, (h0, h0), unroll=16)

    m = mask_ref[:]
    rowmax = jnp.max(m, axis=1, keepdims=True)
    tio = jax.lax.broadcasted_iota(jnp.int32, (B, T), 1)
    tb = jnp.max(jnp.where(m == rowmax, tio, -1), axis=1, keepdims=True)

    cio = jax.lax.broadcasted_iota(jnp.int32, (B, T * B), 1)
    bio = jax.lax.broadcasted_iota(jnp.int32, (B, T * B), 0)
    sel = (cio == tb * B + bio).astype(jnp.float32)
    z_e_ref[:] = jnp.dot(sel, out_ref[:], preferred_element_type=jnp.float32)


def kernel(traj, mask, w_ih_f, w_hh_f, b_ih_f, b_hh_f,
           w_ih_b, w_hh_b, b_ih_b, b_hh_b, codebook):
    del codebook
    B, T, IN = traj.shape
    H = w_hh_f.shape[1]
    x2d = jnp.transpose(traj, (1, 0, 2)).reshape(T * B, IN).astype(jnp.bfloat16)
    scale = jnp.concatenate([jnp.full((2 * H,), 0.5, jnp.float32),
                             jnp.ones((H,), jnp.float32)])
    wih = (jnp.concatenate([w_ih_f.T, w_ih_b.T], axis=1)
           * jnp.concatenate([scale, scale])).astype(jnp.bfloat16)
    zeros_h = jnp.zeros((H,), jnp.float32)
    bg = jnp.concatenate([
        (b_ih_f + jnp.concatenate([b_hh_f[:2 * H], zeros_h])) * scale,
        (b_ih_b + jnp.concatenate([b_hh_b[:2 * H], zeros_h])) * scale,
    ]).reshape(1, -1)
    args = (
        x2d, mask, wih, bg,
        w_hh_f.T.astype(jnp.bfloat16), (0.5 * b_hh_f[2 * H:]).reshape(1, -1),
        w_hh_b.T.astype(jnp.bfloat16), (0.5 * b_hh_b[2 * H:]).reshape(1, -1),
    )
    return pl.pallas_call(
        _bigru_body,
        out_shape=jax.ShapeDtypeStruct((B, 2 * H), jnp.float32),
        scratch_shapes=[
            pltpu.VMEM((T * B, 6 * H), jnp.bfloat16),
            pltpu.VMEM((T * B, 2 * H), jnp.float32),
        ],
    )(*args)
